# baseline (speedup 1.0000x reference)
import functools

import jax
import jax.numpy as jnp
from jax import lax
from jax.experimental import pallas as pl
from jax.experimental.pallas import tpu as pltpu
from jax.experimental.pallas import tpu_sc as plsc

_N = 10000
_E = 320000
_D = 128
_EPS = 1e-5

_NC = 2
_NS = 16
_NW = _NC * _NS

_CHUNK = 80
_NCHUNK = 125
_EPT = _CHUNK * _NCHUNK
_EPAD = _EPT * _NW
_SHIFT = 14
_MASK = (1 << _SHIFT) - 1

_NP = 10112
_ROWS_PER_SUB = _NP // _NS
_PAD_ROW = 10015

_NPD = 10240
_DEG_PER_SUB = _NPD // _NS
_DCHUNK = 64
_DNCHUNK = 160
_DEPT = _DCHUNK * _DNCHUNK
_DEPAD = _DEPT * _NW

_mesh = plsc.VectorSubcoreMesh(core_axis_name="c", subcore_axis_name="s")


def _unpack(pk_v, j, sidx, didx, row):
    for i in range(_CHUNK // 16):
        p = pk_v[j, pl.ds(i * 16, 16)]
        sidx[row, pl.ds(i * 16, 16)] = lax.shift_right_logical(p, _SHIFT)
        didx[row, pl.ds(i * 16, 16)] = lax.bitwise_and(p, _MASK)


@functools.partial(
    pl.kernel,
    out_type=jax.ShapeDtypeStruct((_NC, _NPD), jnp.float32),
    mesh=_mesh,
    scratch_types=[
        pltpu.VMEM_SHARED((_NPD,), jnp.float32),
        pltpu.VMEM((_DNCHUNK // 2, _DCHUNK), jnp.int32),
        pltpu.VMEM((_DCHUNK,), jnp.float32),
    ],
)
def _sc_degree(dstp_hbm, zeros_hbm, ones_hbm, out_hbm, dacc, dst_v, ones_v):
    cid = lax.axis_index("c")
    sid = lax.axis_index("s")
    wid = sid * _NC + cid
    half = _DNCHUNK // 2

    pltpu.sync_copy(ones_hbm, ones_v)
    pltpu.sync_copy(zeros_hbm, dacc.at[pl.ds(sid * _DEG_PER_SUB, _DEG_PER_SUB)])
    plsc.subcore_barrier()

    def chunk(j, carry):
        pltpu.sync_copy(ones_v, dacc.at[dst_v.at[j]], add=True)
        return carry

    for h in range(2):
        pltpu.sync_copy(dstp_hbm.at[wid].at[pl.ds(h * half, half)], dst_v)
        lax.fori_loop(0, half, chunk, 0)
    plsc.subcore_barrier()
    pltpu.sync_copy(
        dacc.at[pl.ds(sid * _DEG_PER_SUB, _DEG_PER_SUB)],
        out_hbm.at[cid].at[pl.ds(sid * _DEG_PER_SUB, _DEG_PER_SUB)],
    )


@functools.partial(
    pl.kernel,
    out_type=jax.ShapeDtypeStruct((_NC, _NP, _D), jnp.float32),
    mesh=_mesh,
    scratch_types=[
        pltpu.VMEM_SHARED((_NP, _D), jnp.float32),
        pltpu.VMEM((_NCHUNK, _CHUNK), jnp.int32),
        pltpu.VMEM((2, _CHUNK), jnp.int32),
        pltpu.VMEM((2, _CHUNK), jnp.int32),
        pltpu.VMEM((_CHUNK, _D // 2), jnp.float32),
        pltpu.VMEM((_CHUNK, _D // 2), jnp.float32),
        pltpu.VMEM((_CHUNK, _D), jnp.float32),
        pltpu.SemaphoreType.DMA,
        pltpu.SemaphoreType.DMA,
    ],
    compiler_params=pltpu.CompilerParams(use_tc_tiling_on_sc=False),
)
def _sc_scatter(z_hbm, pk_hbm, zeros_hbm, out_hbm,
                acc, pk_v, sidx, didx, rows0, rows1, fbuf, sem0, sem1):
    cid = lax.axis_index("c")
    sid = lax.axis_index("s")
    wid = sid * _NC + cid

    pltpu.sync_copy(pk_hbm.at[wid], pk_v)
    pltpu.sync_copy(zeros_hbm, acc.at[pl.ds(sid * _ROWS_PER_SUB, _ROWS_PER_SUB)])
    plsc.subcore_barrier()

    def gather(j, b, buf, sem):
        _unpack(pk_v, j, sidx, didx, b)
        pltpu.make_async_copy(z_hbm.at[sidx.at[b]], buf, sem).start()

    def wait(b, buf, sem):
        pltpu.make_async_copy(z_hbm.at[sidx.at[b]], buf, sem).wait()

    def scatter(b, buf):
        def row(r, carry):
            mask = jnp.full((16,), -65536, jnp.int32)
            shift = jnp.full((16,), 16, jnp.int32)
            for k in range(_D // 32):
                w = lax.bitcast_convert_type(
                    buf[r, pl.ds(16 * k, 16)], jnp.int32)
                lo = lax.bitcast_convert_type(
                    lax.shift_left(w, shift), jnp.float32)
                hi = lax.bitcast_convert_type(
                    lax.bitwise_and(w, mask), jnp.float32)
                fbuf[r, pl.ds(16 * k, 16)] = lo
                fbuf[r, pl.ds(_D // 2 + 16 * k, 16)] = hi
            return carry

        lax.fori_loop(0, _CHUNK, row, 0)
        pltpu.sync_copy(fbuf, acc.at[didx.at[b]], add=True)

    gather(0, 0, rows0, sem0)

    def pair(s, carry):
        j1 = 2 * s + 1
        gather(j1, 1, rows1, sem1)
        wait(0, rows0, sem0)
        scatter(0, rows0)
        gather(j1 + 1, 0, rows0, sem0)
        wait(1, rows1, sem1)
        scatter(1, rows1)
        return carry

    lax.fori_loop(0, (_NCHUNK - 1) // 2, pair, 0)
    wait(0, rows0, sem0)
    scatter(0, rows0)
    plsc.subcore_barrier()
    pltpu.sync_copy(
        acc.at[pl.ds(sid * _ROWS_PER_SUB, _ROWS_PER_SUB)],
        out_hbm.at[cid].at[pl.ds(sid * _ROWS_PER_SUB, _ROWS_PER_SUB)],
    )


_BLK = 1000
_NBLK = _N // _BLK

_row_spec = pl.BlockSpec((_BLK, _D), lambda i: (i, 0))
_col_spec = pl.BlockSpec((_BLK, 1), lambda i: (i, 0))
_w_spec = pl.BlockSpec((_D, _D), lambda i: (0, 0))
_v_spec = pl.BlockSpec((1, _D), lambda i: (0, 0))
_s_spec = pl.BlockSpec((_NC, _BLK, _D), lambda i: (0, i, 0))


def _pack_rows(y):
    b = y.shape[0]
    return jnp.stack([y[:, : _D // 2], y[:, _D // 2:]], axis=-1).astype(
        jnp.bfloat16).reshape(b, _D)


def _tc_in_body(x_ref, w_ref, dinv_ref, o_ref, z_ref):
    u = jnp.dot(dinv_ref[...] * x_ref[...], w_ref[...],
                preferred_element_type=jnp.float32)
    o_ref[...] = u
    z_ref[...] = _pack_rows(u)


_tc_in = pl.pallas_call(
    _tc_in_body,
    grid=(_NBLK,),
    in_specs=[_row_spec, _w_spec, _col_spec],
    out_specs=[_row_spec, _row_spec],
    out_shape=[jax.ShapeDtypeStruct((_N, _D), jnp.float32),
               jax.ShapeDtypeStruct((_N, _D), jnp.bfloat16)],
)


def _tc_mid_body(s_ref, u_ref, dinv_ref, b_ref, g_ref, be_ref, w_ref,
                 o_ref, z_ref):
    dinv = dinv_ref[...]
    t = s_ref[0] + s_ref[1] + u_ref[...]
    z = dinv * t + b_ref[...]
    y = jnp.maximum(z * g_ref[...] + be_ref[...], 0.0)
    u = jnp.dot(dinv * y, w_ref[...], preferred_element_type=jnp.float32)
    o_ref[...] = u
    z_ref[...] = _pack_rows(u)


_tc_mid = pl.pallas_call(
    _tc_mid_body,
    grid=(_NBLK,),
    in_specs=[_s_spec, _row_spec, _col_spec, _v_spec, _v_spec, _v_spec, _w_spec],
    out_specs=[_row_spec, _row_spec],
    out_shape=[jax.ShapeDtypeStruct((_N, _D), jnp.float32),
               jax.ShapeDtypeStruct((_N, _D), jnp.bfloat16)],
)


def _tc_out_body(s_ref, u_ref, dinv_ref, b_ref, g_ref, be_ref, w_ref, rob_ref, o_ref):
    t = s_ref[0] + s_ref[1] + u_ref[...]
    z = dinv_ref[...] * t + b_ref[...]
    y = jnp.maximum(z * g_ref[...] + be_ref[...], 0.0)
    o_ref[...] = jnp.dot(y, w_ref[...], preferred_element_type=jnp.float32) + rob_ref[...]


_tc_out = pl.pallas_call(
    _tc_out_body,
    grid=(_NBLK,),
    in_specs=[_s_spec, _row_spec, _col_spec, _v_spec, _v_spec, _v_spec, _w_spec, _v_spec],
    out_specs=_row_spec,
    out_shape=jax.ShapeDtypeStruct((_N, _D), jnp.float32),
)


def kernel(x, edge_index, W0, b0, g0, be0, W1, b1, g1, be1, W2, b2, g2, be2, roW, rob):
    src = edge_index[0]
    dst = edge_index[1]
    packed = ((src << _SHIFT) | dst).reshape(_NW, _NCHUNK, _CHUNK)
    dstp = jnp.concatenate(
        [dst, jnp.full((_DEPAD - _E,), _PAD_ROW, jnp.int32)]
    ).reshape(_NW, _DNCHUNK, _DCHUNK)

    zeros_deg = jnp.zeros((_DEG_PER_SUB,), jnp.float32)
    ones_deg = jnp.ones((_DCHUNK,), jnp.float32)
    zeros_acc = jnp.zeros((_ROWS_PER_SUB, _D), jnp.float32)

    degp = _sc_degree(dstp, zeros_deg, ones_deg)
    deg = degp[0, :_N] + degp[1, :_N] + 1.0
    dinv = (deg ** -0.5).reshape(_N, 1)

    bn_scale = 1.0 / jnp.sqrt(1.0 + _EPS)
    row = lambda v: v.reshape(1, _D)
    g0s, g1s, g2s = row(g0) * bn_scale, row(g1) * bn_scale, row(g2) * bn_scale

    def pack32(zb):
        return lax.bitcast_convert_type(
            zb.reshape(_N, _D // 2, 2), jnp.float32)

    u, z = _tc_in(x, W0, dinv)
    s = _sc_scatter(pack32(z), packed, zeros_acc)
    u, z = _tc_mid(s, u, dinv, row(b0), g0s, row(be0), W1)
    s = _sc_scatter(pack32(z), packed, zeros_acc)
    u, z = _tc_mid(s, u, dinv, row(b1), g1s, row(be1), W2)
    s = _sc_scatter(pack32(z), packed, zeros_acc)
    return _tc_out(s, u, dinv, row(b2), g2s, row(be2), roW, row(rob))


# speedup vs baseline: 3.3849x; 3.3849x over previous
import functools

import jax
import jax.numpy as jnp
from jax import lax
from jax.experimental import pallas as pl
from jax.experimental.pallas import tpu as pltpu
from jax.experimental.pallas import tpu_sc as plsc

_N = 10000
_E = 320000
_D = 128
_EPS = 1e-5

_NC = 2
_NS = 16
_NW = _NC * _NS

_CHUNK = 80
_NCHUNK = 125
_EPT = _CHUNK * _NCHUNK
_EPAD = _EPT * _NW
_SHIFT = 14
_MASK = (1 << _SHIFT) - 1

_NP = 10112
_ROWS_PER_SUB = _NP // _NS
_PAD_ROW = 10015

_NPD = 10240
_DEG_PER_SUB = _NPD // _NS
_DCHUNK = 64
_DNCHUNK = 160
_DEPT = _DCHUNK * _DNCHUNK
_DEPAD = _DEPT * _NW

_mesh = plsc.VectorSubcoreMesh(core_axis_name="c", subcore_axis_name="s")


def _unpack(pk_v, j, sidx, didx, row):
    for i in range(_CHUNK // 16):
        p = pk_v[j, pl.ds(i * 16, 16)]
        sidx[row, pl.ds(i * 16, 16)] = lax.shift_right_logical(p, _SHIFT)
        didx[row, pl.ds(i * 16, 16)] = lax.bitwise_and(p, _MASK)


@functools.partial(
    pl.kernel,
    out_type=jax.ShapeDtypeStruct((_NC, _NPD), jnp.float32),
    mesh=_mesh,
    scratch_types=[
        pltpu.VMEM_SHARED((_NPD,), jnp.float32),
        pltpu.VMEM((_DNCHUNK // 2, _DCHUNK), jnp.int32),
        pltpu.VMEM((_DCHUNK,), jnp.float32),
    ],
)
def _sc_degree(dstp_hbm, zeros_hbm, ones_hbm, out_hbm, dacc, dst_v, ones_v):
    cid = lax.axis_index("c")
    sid = lax.axis_index("s")
    wid = sid * _NC + cid
    half = _DNCHUNK // 2

    pltpu.sync_copy(ones_hbm, ones_v)
    pltpu.sync_copy(zeros_hbm, dacc.at[pl.ds(sid * _DEG_PER_SUB, _DEG_PER_SUB)])
    plsc.subcore_barrier()

    def chunk(j, carry):
        pltpu.sync_copy(ones_v, dacc.at[dst_v.at[j]], add=True)
        return carry

    for h in range(2):
        pltpu.sync_copy(dstp_hbm.at[wid].at[pl.ds(h * half, half)], dst_v)
        lax.fori_loop(0, half, chunk, 0)
    plsc.subcore_barrier()
    pltpu.sync_copy(
        dacc.at[pl.ds(sid * _DEG_PER_SUB, _DEG_PER_SUB)],
        out_hbm.at[cid].at[pl.ds(sid * _DEG_PER_SUB, _DEG_PER_SUB)],
    )


@functools.partial(
    pl.kernel,
    out_type=jax.ShapeDtypeStruct((_NC, _NP, _D), jnp.float32),
    mesh=_mesh,
    scratch_types=[
        pltpu.VMEM_SHARED((_NP, _D), jnp.float32),
        pltpu.VMEM((_NCHUNK, _CHUNK), jnp.int32),
        pltpu.VMEM((2, _CHUNK), jnp.int32),
        pltpu.VMEM((2, _CHUNK), jnp.int32),
        pltpu.VMEM((_CHUNK, _D), jnp.float32),
        pltpu.VMEM((_CHUNK, _D), jnp.float32),
        pltpu.SemaphoreType.DMA,
        pltpu.SemaphoreType.DMA,
    ],
)
def _sc_scatter(z_hbm, pk_hbm, zeros_hbm, out_hbm,
                acc, pk_v, sidx, didx, rows0, rows1, sem0, sem1):
    cid = lax.axis_index("c")
    sid = lax.axis_index("s")
    wid = sid * _NC + cid

    pltpu.sync_copy(pk_hbm.at[wid], pk_v)
    pltpu.sync_copy(zeros_hbm, acc.at[pl.ds(sid * _ROWS_PER_SUB, _ROWS_PER_SUB)])
    plsc.subcore_barrier()

    def gather(j, b, buf, sem):
        _unpack(pk_v, j, sidx, didx, b)
        pltpu.make_async_copy(z_hbm.at[sidx.at[b]], buf, sem).start()

    def wait(b, buf, sem):
        pltpu.make_async_copy(z_hbm.at[sidx.at[b]], buf, sem).wait()

    def scatter(b, buf):
        pltpu.sync_copy(buf, acc.at[didx.at[b]], add=True)

    gather(0, 0, rows0, sem0)

    def pair(s, carry):
        j1 = 2 * s + 1
        gather(j1, 1, rows1, sem1)
        wait(0, rows0, sem0)
        scatter(0, rows0)
        gather(j1 + 1, 0, rows0, sem0)
        wait(1, rows1, sem1)
        scatter(1, rows1)
        return carry

    lax.fori_loop(0, (_NCHUNK - 1) // 2, pair, 0)
    wait(0, rows0, sem0)
    scatter(0, rows0)
    plsc.subcore_barrier()
    pltpu.sync_copy(
        acc.at[pl.ds(sid * _ROWS_PER_SUB, _ROWS_PER_SUB)],
        out_hbm.at[cid].at[pl.ds(sid * _ROWS_PER_SUB, _ROWS_PER_SUB)],
    )


_BLK = 1000
_NBLK = _N // _BLK

_row_spec = pl.BlockSpec((_BLK, _D), lambda i: (i, 0))
_col_spec = pl.BlockSpec((_BLK, 1), lambda i: (i, 0))
_w_spec = pl.BlockSpec((_D, _D), lambda i: (0, 0))
_v_spec = pl.BlockSpec((1, _D), lambda i: (0, 0))
_s_spec = pl.BlockSpec((_NC, _BLK, _D), lambda i: (0, i, 0))


def _tc_in_body(x_ref, w_ref, dinv_ref, o_ref):
    o_ref[...] = jnp.dot(dinv_ref[...] * x_ref[...], w_ref[...],
                         preferred_element_type=jnp.float32)


_tc_in = pl.pallas_call(
    _tc_in_body,
    grid=(_NBLK,),
    in_specs=[_row_spec, _w_spec, _col_spec],
    out_specs=_row_spec,
    out_shape=jax.ShapeDtypeStruct((_N, _D), jnp.float32),
)


def _tc_mid_body(s_ref, u_ref, dinv_ref, b_ref, g_ref, be_ref, w_ref, o_ref):
    dinv = dinv_ref[...]
    t = s_ref[0] + s_ref[1] + u_ref[...]
    z = dinv * t + b_ref[...]
    y = jnp.maximum(z * g_ref[...] + be_ref[...], 0.0)
    o_ref[...] = jnp.dot(dinv * y, w_ref[...], preferred_element_type=jnp.float32)


_tc_mid = pl.pallas_call(
    _tc_mid_body,
    grid=(_NBLK,),
    in_specs=[_s_spec, _row_spec, _col_spec, _v_spec, _v_spec, _v_spec, _w_spec],
    out_specs=_row_spec,
    out_shape=jax.ShapeDtypeStruct((_N, _D), jnp.float32),
)


def _tc_out_body(s_ref, u_ref, dinv_ref, b_ref, g_ref, be_ref, w_ref, rob_ref, o_ref):
    t = s_ref[0] + s_ref[1] + u_ref[...]
    z = dinv_ref[...] * t + b_ref[...]
    y = jnp.maximum(z * g_ref[...] + be_ref[...], 0.0)
    o_ref[...] = jnp.dot(y, w_ref[...], preferred_element_type=jnp.float32) + rob_ref[...]


_tc_out = pl.pallas_call(
    _tc_out_body,
    grid=(_NBLK,),
    in_specs=[_s_spec, _row_spec, _col_spec, _v_spec, _v_spec, _v_spec, _w_spec, _v_spec],
    out_specs=_row_spec,
    out_shape=jax.ShapeDtypeStruct((_N, _D), jnp.float32),
)


def kernel(x, edge_index, W0, b0, g0, be0, W1, b1, g1, be1, W2, b2, g2, be2, roW, rob):
    src = edge_index[0]
    dst = edge_index[1]
    packed = ((src << _SHIFT) | dst).reshape(_NW, _NCHUNK, _CHUNK)
    dstp = jnp.concatenate(
        [dst, jnp.full((_DEPAD - _E,), _PAD_ROW, jnp.int32)]
    ).reshape(_NW, _DNCHUNK, _DCHUNK)

    zeros_deg = jnp.zeros((_DEG_PER_SUB,), jnp.float32)
    ones_deg = jnp.ones((_DCHUNK,), jnp.float32)
    zeros_acc = jnp.zeros((_ROWS_PER_SUB, _D), jnp.float32)

    degp = _sc_degree(dstp, zeros_deg, ones_deg)
    deg = degp[0, :_N] + degp[1, :_N] + 1.0
    dinv = (deg ** -0.5).reshape(_N, 1)

    bn_scale = 1.0 / jnp.sqrt(1.0 + _EPS)
    row = lambda v: v.reshape(1, _D)
    g0s, g1s, g2s = row(g0) * bn_scale, row(g1) * bn_scale, row(g2) * bn_scale

    u = _tc_in(x, W0, dinv)
    s = _sc_scatter(u, packed, zeros_acc)
    u = _tc_mid(s, u, dinv, row(b0), g0s, row(be0), W1)
    s = _sc_scatter(u, packed, zeros_acc)
    u = _tc_mid(s, u, dinv, row(b1), g1s, row(be1), W2)
    s = _sc_scatter(u, packed, zeros_acc)
    return _tc_out(s, u, dinv, row(b2), g2s, row(be2), roW, row(rob))
